# speedup vs baseline: 16.5044x; 16.5044x over previous
import jax
import jax.numpy as jnp
from jax.experimental import pallas as pl
from jax.experimental.pallas import tpu as pltpu

B = 16
TOTAL = 16384
D = 256
SEG = TOTAL // B


def _seg_kernel(inv_count_ref, x_ref, out_ref):
    i = pl.program_id(0)
    xb = x_ref[...]
    inv = inv_count_ref[i]
    mean = jnp.sum(xb, axis=0, keepdims=True) * inv
    mn = jnp.min(xb, axis=0, keepdims=True)
    mx = jnp.max(xb, axis=0, keepdims=True)
    out_ref[:, 0:D] = jnp.broadcast_to(mean, (SEG, D))
    out_ref[:, D:2 * D] = jnp.broadcast_to(mn, (SEG, D))
    out_ref[:, 2 * D:3 * D] = jnp.broadcast_to(mx, (SEG, D))
    out_ref[:, 3 * D:4 * D] = xb


def kernel(x_data, row_splits):
    counts = (row_splits[1:] - row_splits[:-1]).astype(jnp.float32)
    inv_counts = 1.0 / counts
    grid = (B,)
    return pl.pallas_call(
        _seg_kernel,
        grid_spec=pltpu.PrefetchScalarGridSpec(
            num_scalar_prefetch=1,
            grid=grid,
            in_specs=[pl.BlockSpec((SEG, D), lambda i, *_: (i, 0))],
            out_specs=pl.BlockSpec((SEG, 4 * D), lambda i, *_: (i, 0)),
        ),
        out_shape=jax.ShapeDtypeStruct((TOTAL, 4 * D), jnp.float32),
    )(inv_counts, x_data)
